# Initial kernel scaffold; baseline (speedup 1.0000x reference)
import jax
import jax.numpy as jnp
from jax.experimental import pallas as pl


def kernel(xyz, points, W0, b0, g0, beta0, W1, b1, g1, beta1, W2, b2, g2, beta2):
    raise NotImplementedError("write your pallas kernel here")


# speedup vs baseline: 17.9750x; 17.9750x over previous
import functools

import numpy as np
import jax
import jax.numpy as jnp
from jax import lax
from jax.experimental import pallas as pl
from jax.experimental.pallas import tpu as pltpu
from jax.experimental.pallas import tpu_sc as plsc

_B = 8
_N = 4096
_S = 1024
_K = 32
_DPAD = 144
_EPS = 1e-5
_R2 = np.float32(0.4 ** 2)
_M = _B * _S * _K
_INV_M = np.float32(1.0 / _M)

_NW = 32
_RPW = _B * _S // _NW
_CPB = _N // 16


def _fps_body(x_ref, y_ref, z_ref, ox_ref, oy_ref, oz_ref):
    X = x_ref[...]
    Y = y_ref[...]
    Z = z_ref[...]
    iota = lax.broadcasted_iota(jnp.int32, (_B, _N), 1)

    def body(i, carry):
        dist, far = carry
        msk = iota == far
        zero = jnp.zeros_like(X)
        cx = jnp.sum(jnp.where(msk, X, zero), axis=1, keepdims=True)
        cy = jnp.sum(jnp.where(msk, Y, zero), axis=1, keepdims=True)
        cz = jnp.sum(jnp.where(msk, Z, zero), axis=1, keepdims=True)
        ox_ref[pl.ds(i, 1)] = cx[None]
        oy_ref[pl.ds(i, 1)] = cy[None]
        oz_ref[pl.ds(i, 1)] = cz[None]
        dx = X - cx
        dy = Y - cy
        dz = Z - cz
        d = (dx * dx + dy * dy) + dz * dz
        dist = jnp.minimum(dist, d)
        m = jnp.max(dist, axis=1, keepdims=True)
        far = jnp.min(jnp.where(dist == m, iota, _N), axis=1, keepdims=True)
        return dist, far

    dist0 = jnp.full((_B, _N), 1e10, jnp.float32)
    far0 = jnp.zeros((_B, 1), jnp.int32)
    lax.fori_loop(0, _S, body, (dist0, far0))


def _table_body(pts_ref, o_ref):
    o_ref[...] = jnp.transpose(pts_ref[0], (1, 0))


def _bf16r(v):
    u = plsc.bitcast(v, jnp.uint32)
    lsb = jnp.bitwise_and(lax.shift_right_logical(u, jnp.uint32(16)),
                          jnp.uint32(1))
    u = u + (jnp.uint32(0x7FFF) + lsb)
    u = jnp.bitwise_and(u, jnp.uint32(0xFFFF0000))
    return plsc.bitcast(u, jnp.float32)


def _sc_body(xf, yf, zf, qxf, qyf, qzf, tab, out, oxyz,
             px, py, pz, pxr, pyr, pzr, sp, qx, qy, qz, slots, gidx, gxb,
             rb0, rb1, sem0, sem1):
    cid = lax.axis_index("c")
    sid = lax.axis_index("s")
    w = sid * 2 + cid
    b = w // 4
    s0 = (w % 4) * _RPW
    boff = b * _N

    pltpu.sync_copy(xf.at[pl.ds(b * _N, _N)], px)
    pltpu.sync_copy(yf.at[pl.ds(b * _N, _N)], py)
    pltpu.sync_copy(zf.at[pl.ds(b * _N, _N)], pz)
    pltpu.sync_copy(qxf.at[pl.ds(b * _S + s0, _RPW)], qx)
    pltpu.sync_copy(qyf.at[pl.ds(b * _S + s0, _RPW)], qy)
    pltpu.sync_copy(qzf.at[pl.ds(b * _S + s0, _RPW)], qz)

    def spbody(i, _):
        pxv = px[pl.ds(i * 16, 16)]
        pyv = py[pl.ds(i * 16, 16)]
        pzv = pz[pl.ds(i * 16, 16)]
        sp[pl.ds(i * 16, 16)] = (pxv * pxv + pyv * pyv) + pzv * pzv
        pxr[pl.ds(i * 16, 16)] = _bf16r(pxv)
        pyr[pl.ds(i * 16, 16)] = _bf16r(pyv)
        pzr[pl.ds(i * 16, 16)] = _bf16r(pzv)
        return 0

    lax.fori_loop(0, _CPB, spbody, 0)

    iota16 = lax.iota(jnp.int32, 16)

    def _splat(vec, lane):
        zero = jnp.zeros_like(vec)
        s = jnp.sum(jnp.where(iota16 == lane, vec, zero))
        return jnp.full((16,), s, vec.dtype)

    def rowbody(r, _):
        g16 = (r // 16) * 16
        lane = r % 16
        qxs = _splat(qx[pl.ds(g16, 16)], lane)
        qys = _splat(qy[pl.ds(g16, 16)], lane)
        qzs = _splat(qz[pl.ds(g16, 16)], lane)
        sq = (qxs * qxs + qys * qys) + qzs * qzs
        qxr = _bf16r(qxs)
        qyr = _bf16r(qys)
        qzr = _bf16r(qzs)

        def cond(st):
            cnt, c = st
            return (cnt < _K) & (c < _CPB)

        def step(st):
            cnt, c = st
            base = c * 16
            pxv = pxr[pl.ds(base, 16)]
            pyv = pyr[pl.ds(base, 16)]
            pzv = pzr[pl.ds(base, 16)]
            spv = sp[pl.ds(base, 16)]
            m3 = (qxr * pxv + qyr * pyv) + qzr * pzv
            d = (-2.0 * m3 + sq) + spv
            msk = d <= _R2
            mi = msk.astype(jnp.int32)
            slot = (cnt + jnp.cumsum(mi)) - 1
            nvec = base + iota16
            plsc.store_scatter(slots, [slot], nvec, mask=msk)
            return cnt + jnp.sum(mi), c + 1

        cnt, _c = lax.while_loop(cond, step, (jnp.int32(0), jnp.int32(0)))
        first = _splat(slots[pl.ds(0, 16)], 0)
        v0 = jnp.where(iota16 < cnt, slots[pl.ds(0, 16)], first)
        v1 = jnp.where(iota16 + 16 < cnt, slots[pl.ds(16, 16)], first)
        gidx[pl.ds(r * _K, 16)] = v0 + boff
        gidx[pl.ds(r * _K + 16, 16)] = v1 + boff
        r8 = (r % 8) * _K
        for kk, vv in ((0, v0), (1, v1)):
            rows = r8 + 16 * kk + iota16
            for cc, plane in ((0, px), (1, py), (2, pz)):
                coords = plsc.load_gather(plane, [vv])
                plsc.store_scatter(gxb, [rows, jnp.full((16,), cc, jnp.int32)],
                                   coords)
        @pl.when(r % 8 == 7)
        def _():
            pltpu.sync_copy(
                gxb, oxyz.at[pl.ds((w * _RPW + r - 7) * _K, 8 * _K)])
        return 0

    lax.fori_loop(0, _RPW, rowbody, 0)

    ob = w * (_RPW * _K)

    def gbody(j, _):
        isl = gidx.at[pl.ds(j * 128, 128)]
        pltpu.async_copy(tab.at[isl], rb0, sem0).wait()
        pltpu.sync_copy(rb0, out.at[pl.ds(ob + j * 128, 128)])
        return 0

    lax.fori_loop(0, _RPW * _K // 128, gbody, 0)


def _mmb(x, wh):
    dn = (((1,), (0,)), ((), ()))
    return lax.dot_general(x.astype(jnp.bfloat16), wh, dn,
                           preferred_element_type=jnp.float32)


def _stats_update(st_ref, y, t):
    @pl.when(t == 0)
    def _():
        st_ref[...] = jnp.zeros_like(st_ref)

    s1 = jnp.sum(y, axis=0).reshape(1, -1)
    s2 = jnp.sum(y * y, axis=0).reshape(1, -1)
    pad = jnp.zeros((6, y.shape[1]), jnp.float32)
    st_ref[...] = st_ref[...] + jnp.concatenate([s1, s2, pad], axis=0)


def _mlp0_body(gf_ref, gx_ref, qx_ref, qy_ref, qz_ref, wh_ref,
               bb_ref, y_ref, st_ref):
    t = pl.program_id(0)
    b = t // 16
    col = lax.broadcasted_iota(jnp.int32, (64, 8), 1)
    zq = jnp.zeros((64, 8), jnp.float32)

    def pick(ref):
        return jnp.sum(jnp.where(col == b, ref[...], zq), axis=1,
                       keepdims=True)

    x = jnp.concatenate(
        [gx_ref[...][:, 0:3], gf_ref[...],
         jnp.zeros((64 * _K, _DPAD - 131), jnp.float32)], axis=1)
    q144 = jnp.concatenate(
        [pick(qx_ref), pick(qy_ref), pick(qz_ref),
         jnp.zeros((64, _DPAD - 3), jnp.float32)], axis=1)
    x3 = x.reshape(64, _K, _DPAD) - q144[:, None, :]
    x2 = x3.reshape(64 * _K, _DPAD)
    y = _mmb(x2, wh_ref[...]) + bb_ref[...][0:1, :]
    y_ref[...] = y
    _stats_update(st_ref, y, t)


def _affine_relu(y, st_ref, g_ref, be_ref):
    st = st_ref[...]
    mean = st[0:1, :] * _INV_M
    var = st[1:2, :] * _INV_M - mean * mean
    den = jnp.sqrt(var + _EPS)
    h = (y - mean) / den * g_ref[...][0:1, :] + be_ref[...][0:1, :]
    return jnp.maximum(h, 0.0)


def _mlp_body(y_ref, st_ref, g_ref, be_ref, wh_ref, bb_ref,
              o_ref, sto_ref):
    t = pl.program_id(0)
    h = _affine_relu(y_ref[...], st_ref, g_ref, be_ref)
    o = _mmb(h, wh_ref[...]) + bb_ref[...][0:1, :]
    o_ref[...] = o
    _stats_update(sto_ref, o, t)


def _final_body(y_ref, st_ref, g_ref, be_ref, o_ref):
    h = _affine_relu(y_ref[...], st_ref, g_ref, be_ref)
    mx = jnp.max(h.reshape(128, _K, 256), axis=1)
    o_ref[...] = jnp.transpose(mx, (1, 0))[None]


def _bc(v):
    return jnp.broadcast_to(v.reshape(1, -1), (8, v.shape[0]))


def kernel(xyz, points, W0, b0, g0, beta0, W1, b1, g1, beta1,
           W2, b2, g2, beta2):
    f32 = jnp.float32
    xpl = xyz[:, 0, :]
    ypl = xyz[:, 1, :]
    zpl = xyz[:, 2, :]

    ox3, oy3, oz3 = pl.pallas_call(
        _fps_body,
        out_shape=[jax.ShapeDtypeStruct((_S, _B, 1), f32)] * 3,
    )(xpl, ypl, zpl)
    qxt = ox3.reshape(_S, _B)
    qyt = oy3.reshape(_S, _B)
    qzt = oz3.reshape(_S, _B)
    ox = qxt.T
    oy = qyt.T
    oz = qzt.T

    table = pl.pallas_call(
        _table_body,
        grid=(_B, _N // 256),
        in_specs=[
            pl.BlockSpec((1, 128, 256), lambda b, j: (b, 0, j)),
        ],
        out_specs=pl.BlockSpec((256, 128), lambda b, j: (b * (_N // 256) + j, 0)),
        out_shape=jax.ShapeDtypeStruct((_B * _N, 128), f32),
        compiler_params=pltpu.CompilerParams(
            dimension_semantics=("arbitrary", "arbitrary")),
    )(points)

    mesh = plsc.VectorSubcoreMesh(core_axis_name="c", subcore_axis_name="s")
    sc = functools.partial(
        pl.kernel,
        out_type=[
            jax.ShapeDtypeStruct((_M, 128), f32),
            jax.ShapeDtypeStruct((_M, 16), f32),
        ],
        mesh=mesh,
        compiler_params=pltpu.CompilerParams(needs_layout_passes=False),
        scratch_types=[
            pltpu.VMEM((_N,), f32), pltpu.VMEM((_N,), f32),
            pltpu.VMEM((_N,), f32), pltpu.VMEM((_N,), f32),
            pltpu.VMEM((_N,), f32), pltpu.VMEM((_N,), f32),
            pltpu.VMEM((_N,), f32),
            pltpu.VMEM((_RPW,), f32), pltpu.VMEM((_RPW,), f32),
            pltpu.VMEM((_RPW,), f32),
            pltpu.VMEM((48,), jnp.int32),
            pltpu.VMEM((_RPW * _K,), jnp.int32),
            pltpu.VMEM((8 * _K, 16), f32),
            pltpu.VMEM((128, 128), f32), pltpu.VMEM((128, 128), f32),
            pltpu.SemaphoreType.DMA, pltpu.SemaphoreType.DMA,
        ],
    )(_sc_body)
    gfeat, gxyz = sc(xpl.reshape(-1), ypl.reshape(-1), zpl.reshape(-1),
                     ox.reshape(-1), oy.reshape(-1), oz.reshape(-1), table)

    w0h = jnp.pad(W0.T, ((0, _DPAD - 131), (0, 0))).astype(jnp.bfloat16)
    w1h = W1.T.astype(jnp.bfloat16)
    w2h = W2.T.astype(jnp.bfloat16)

    n_blk = _M // 2048
    cp = pltpu.CompilerParams(dimension_semantics=("arbitrary",))
    y0, st0 = pl.pallas_call(
        _mlp0_body,
        grid=(n_blk,),
        in_specs=[
            pl.BlockSpec((2048, 128), lambda t: (t, 0)),
            pl.BlockSpec((2048, 16), lambda t: (t, 0)),
            pl.BlockSpec((64, 8), lambda t: (t % 16, 0)),
            pl.BlockSpec((64, 8), lambda t: (t % 16, 0)),
            pl.BlockSpec((64, 8), lambda t: (t % 16, 0)),
            pl.BlockSpec((_DPAD, 128), lambda t: (0, 0)),
            pl.BlockSpec((8, 128), lambda t: (0, 0)),
        ],
        out_specs=[
            pl.BlockSpec((2048, 128), lambda t: (t, 0)),
            pl.BlockSpec((8, 128), lambda t: (0, 0)),
        ],
        out_shape=[
            jax.ShapeDtypeStruct((_M, 128), f32),
            jax.ShapeDtypeStruct((8, 128), f32),
        ],
        compiler_params=cp,
    )(gfeat, gxyz, qxt, qyt, qzt, w0h, _bc(b0))

    def mid_layer(y, st, g, be, wh, bb, cout):
        return pl.pallas_call(
            _mlp_body,
            grid=(n_blk,),
            in_specs=[
                pl.BlockSpec((2048, 128), lambda t: (t, 0)),
                pl.BlockSpec((8, 128), lambda t: (0, 0)),
                pl.BlockSpec((8, 128), lambda t: (0, 0)),
                pl.BlockSpec((8, 128), lambda t: (0, 0)),
                pl.BlockSpec((128, cout), lambda t: (0, 0)),
                pl.BlockSpec((8, cout), lambda t: (0, 0)),
            ],
            out_specs=[
                pl.BlockSpec((2048, cout), lambda t: (t, 0)),
                pl.BlockSpec((8, cout), lambda t: (0, 0)),
            ],
            out_shape=[
                jax.ShapeDtypeStruct((_M, cout), f32),
                jax.ShapeDtypeStruct((8, cout), f32),
            ],
            compiler_params=cp,
        )(y, st, _bc(g), _bc(be), wh, _bc(bb))

    y1, st1 = mid_layer(y0, st0, g0, beta0, w1h, b1, 128)
    y2, st2 = mid_layer(y1, st1, g1, beta1, w2h, b2, 256)

    new_points = pl.pallas_call(
        _final_body,
        grid=(_M // 4096,),
        in_specs=[
            pl.BlockSpec((4096, 256), lambda t: (t, 0)),
            pl.BlockSpec((8, 256), lambda t: (0, 0)),
            pl.BlockSpec((8, 256), lambda t: (0, 0)),
            pl.BlockSpec((8, 256), lambda t: (0, 0)),
        ],
        out_specs=pl.BlockSpec((1, 256, 128), lambda t: (t // 8, 0, t % 8)),
        out_shape=jax.ShapeDtypeStruct((_B, 256, _S), f32),
        compiler_params=cp,
    )(y2, st2, _bc(g2), _bc(beta2))

    new_xyz = jnp.stack([ox, oy, oz], axis=1)
    return new_xyz, new_points


# speedup vs baseline: 19.8645x; 1.1051x over previous
import functools

import numpy as np
import jax
import jax.numpy as jnp
from jax import lax
from jax.experimental import pallas as pl
from jax.experimental.pallas import tpu as pltpu
from jax.experimental.pallas import tpu_sc as plsc

_B = 8
_N = 4096
_S = 1024
_K = 32
_DPAD = 144
_EPS = 1e-5
_R2 = np.float32(0.4 ** 2)
_M = _B * _S * _K
_INV_M = np.float32(1.0 / _M)

_BLK = 4096
_RQ = _BLK // _K
_NW = 32
_RPW = _B * _S // _NW
_CPB = _N // 16


def _tree(x, op2, fin):
    n = x.shape[1]
    while n > 128:
        h = n // 2
        x = op2(x[:, :h], x[:, h:])
        n = h
    return fin(x, axis=1, keepdims=True)


def _fps_body(x_ref, y_ref, z_ref, ox_ref, oy_ref, oz_ref):
    X = x_ref[...]
    Y = y_ref[...]
    Z = z_ref[...]
    iota = lax.broadcasted_iota(jnp.int32, (_B, _N), 1)

    def body(i, carry):
        dist, far = carry
        msk = iota == far
        zero = jnp.zeros_like(X)
        cx = _tree(jnp.where(msk, X, zero), jnp.add, jnp.sum)
        cy = _tree(jnp.where(msk, Y, zero), jnp.add, jnp.sum)
        cz = _tree(jnp.where(msk, Z, zero), jnp.add, jnp.sum)
        ox_ref[pl.ds(i, 1)] = cx[None]
        oy_ref[pl.ds(i, 1)] = cy[None]
        oz_ref[pl.ds(i, 1)] = cz[None]
        dx = X - cx
        dy = Y - cy
        dz = Z - cz
        d = (dx * dx + dy * dy) + dz * dz
        dist = jnp.minimum(dist, d)
        m = _tree(dist, jnp.maximum, jnp.max)
        far = _tree(jnp.where(dist == m, iota, _N), jnp.minimum, jnp.min)
        return dist, far

    dist0 = jnp.full((_B, _N), 1e10, jnp.float32)
    far0 = jnp.zeros((_B, 1), jnp.int32)
    lax.fori_loop(0, _S, body, (dist0, far0))


def _table_body(pts_ref, o_ref):
    o_ref[...] = jnp.transpose(pts_ref[0], (1, 0))


def _bf16r(v):
    u = plsc.bitcast(v, jnp.uint32)
    lsb = jnp.bitwise_and(lax.shift_right_logical(u, jnp.uint32(16)),
                          jnp.uint32(1))
    u = u + (jnp.uint32(0x7FFF) + lsb)
    u = jnp.bitwise_and(u, jnp.uint32(0xFFFF0000))
    return plsc.bitcast(u, jnp.float32)


def _sc_body(xf, yf, zf, qxf, qyf, qzf, tab, out, oxyz,
             px, py, pz, pxr, pyr, pzr, sp, qx, qy, qz, slots, gidx, gxb,
             rb0, rb1, sem0, sem1):
    cid = lax.axis_index("c")
    sid = lax.axis_index("s")
    w = sid * 2 + cid
    b = w // 4
    s0 = (w % 4) * _RPW
    boff = b * _N

    pltpu.sync_copy(xf.at[pl.ds(b * _N, _N)], px)
    pltpu.sync_copy(yf.at[pl.ds(b * _N, _N)], py)
    pltpu.sync_copy(zf.at[pl.ds(b * _N, _N)], pz)
    pltpu.sync_copy(qxf.at[pl.ds(b * _S + s0, _RPW)], qx)
    pltpu.sync_copy(qyf.at[pl.ds(b * _S + s0, _RPW)], qy)
    pltpu.sync_copy(qzf.at[pl.ds(b * _S + s0, _RPW)], qz)

    def spbody(i, _):
        pxv = px[pl.ds(i * 16, 16)]
        pyv = py[pl.ds(i * 16, 16)]
        pzv = pz[pl.ds(i * 16, 16)]
        sp[pl.ds(i * 16, 16)] = (pxv * pxv + pyv * pyv) + pzv * pzv
        pxr[pl.ds(i * 16, 16)] = _bf16r(pxv)
        pyr[pl.ds(i * 16, 16)] = _bf16r(pyv)
        pzr[pl.ds(i * 16, 16)] = _bf16r(pzv)
        return 0

    lax.fori_loop(0, _CPB, spbody, 0)

    iota16 = lax.iota(jnp.int32, 16)

    def _splat(vec, lane):
        zero = jnp.zeros_like(vec)
        s = jnp.sum(jnp.where(iota16 == lane, vec, zero))
        return jnp.full((16,), s, vec.dtype)

    def rowbody(r, _):
        g16 = (r // 16) * 16
        lane = r % 16
        qxs = _splat(qx[pl.ds(g16, 16)], lane)
        qys = _splat(qy[pl.ds(g16, 16)], lane)
        qzs = _splat(qz[pl.ds(g16, 16)], lane)
        sq = (qxs * qxs + qys * qys) + qzs * qzs
        qxr = _bf16r(qxs)
        qyr = _bf16r(qys)
        qzr = _bf16r(qzs)

        def cond(st):
            cnt, c = st
            return (cnt < _K) & (c < _CPB)

        def step(st):
            cnt, c = st
            base = c * 16
            pxv = pxr[pl.ds(base, 16)]
            pyv = pyr[pl.ds(base, 16)]
            pzv = pzr[pl.ds(base, 16)]
            spv = sp[pl.ds(base, 16)]
            m3 = (qxr * pxv + qyr * pyv) + qzr * pzv
            d = (-2.0 * m3 + sq) + spv
            msk = d <= _R2
            mi = msk.astype(jnp.int32)
            slot = (cnt + jnp.cumsum(mi)) - 1
            nvec = base + iota16
            plsc.store_scatter(slots, [slot], nvec, mask=msk)
            return cnt + jnp.sum(mi), c + 1

        cnt, _c = lax.while_loop(cond, step, (jnp.int32(0), jnp.int32(0)))
        first = _splat(slots[pl.ds(0, 16)], 0)
        v0 = jnp.where(iota16 < cnt, slots[pl.ds(0, 16)], first)
        v1 = jnp.where(iota16 + 16 < cnt, slots[pl.ds(16, 16)], first)
        gidx[pl.ds(r * _K, 16)] = v0 + boff
        gidx[pl.ds(r * _K + 16, 16)] = v1 + boff
        r8 = (r % 8) * _K
        for kk, vv in ((0, v0), (1, v1)):
            rows = r8 + 16 * kk + iota16
            for cc, plane in ((0, px), (1, py), (2, pz)):
                coords = plsc.load_gather(plane, [vv])
                plsc.store_scatter(gxb, [rows, jnp.full((16,), cc, jnp.int32)],
                                   coords)
        @pl.when(r % 8 == 7)
        def _():
            pltpu.sync_copy(
                gxb, oxyz.at[pl.ds((w * _RPW + r - 7) * _K, 8 * _K)])
        return 0

    lax.fori_loop(0, _RPW, rowbody, 0)

    ob = w * (_RPW * _K)

    def gbody(j, _):
        isl = gidx.at[pl.ds(j * 128, 128)]
        pltpu.async_copy(tab.at[isl], rb0, sem0).wait()
        pltpu.sync_copy(rb0, out.at[pl.ds(ob + j * 128, 128)])
        return 0

    lax.fori_loop(0, _RPW * _K // 128, gbody, 0)


def _mmb(x, wh):
    dn = (((1,), (0,)), ((), ()))
    return lax.dot_general(x.astype(jnp.bfloat16), wh, dn,
                           preferred_element_type=jnp.float32)


def _stats_update(st_ref, y, t):
    @pl.when(t == 0)
    def _():
        st_ref[...] = jnp.zeros_like(st_ref)

    s1 = jnp.sum(y, axis=0).reshape(1, -1)
    s2 = jnp.sum(y * y, axis=0).reshape(1, -1)
    pad = jnp.zeros((6, y.shape[1]), jnp.float32)
    st_ref[...] = st_ref[...] + jnp.concatenate([s1, s2, pad], axis=0)


def _mlp0_body(gf_ref, gx_ref, qx_ref, qy_ref, qz_ref, wh_ref,
               bb_ref, y_ref, st_ref):
    t = pl.program_id(0)
    b = t // (_S * _K // _BLK)
    col = lax.broadcasted_iota(jnp.int32, (_RQ, 8), 1)
    zq = jnp.zeros((_RQ, 8), jnp.float32)

    def pick(ref):
        return jnp.sum(jnp.where(col == b, ref[...], zq), axis=1,
                       keepdims=True)

    x = jnp.concatenate(
        [gx_ref[...][:, 0:3], gf_ref[...],
         jnp.zeros((_RQ * _K, _DPAD - 131), jnp.float32)], axis=1)
    q144 = jnp.concatenate(
        [pick(qx_ref), pick(qy_ref), pick(qz_ref),
         jnp.zeros((_RQ, _DPAD - 3), jnp.float32)], axis=1)
    x3 = x.reshape(_RQ, _K, _DPAD) - q144[:, None, :]
    x2 = x3.reshape(_RQ * _K, _DPAD)
    y = _mmb(x2, wh_ref[...]) + bb_ref[...][0:1, :]
    y_ref[...] = y
    _stats_update(st_ref, y, t)


def _affine_relu(y, st_ref, g_ref, be_ref):
    st = st_ref[...]
    mean = st[0:1, :] * _INV_M
    var = st[1:2, :] * _INV_M - mean * mean
    den = jnp.sqrt(var + _EPS)
    h = (y - mean) / den * g_ref[...][0:1, :] + be_ref[...][0:1, :]
    return jnp.maximum(h, 0.0)


def _mlp_body(y_ref, st_ref, g_ref, be_ref, wh_ref, bb_ref,
              o_ref, sto_ref):
    t = pl.program_id(0)
    h = _affine_relu(y_ref[...], st_ref, g_ref, be_ref)
    o = _mmb(h, wh_ref[...]) + bb_ref[...][0:1, :]
    o_ref[...] = o
    _stats_update(sto_ref, o, t)


def _final_body(y_ref, st_ref, g_ref, be_ref, o_ref):
    h = _affine_relu(y_ref[...], st_ref, g_ref, be_ref)
    mx = jnp.max(h.reshape(128, _K, 256), axis=1)
    o_ref[...] = jnp.transpose(mx, (1, 0))[None]


def _bc(v):
    return jnp.broadcast_to(v.reshape(1, -1), (8, v.shape[0]))


def kernel(xyz, points, W0, b0, g0, beta0, W1, b1, g1, beta1,
           W2, b2, g2, beta2):
    f32 = jnp.float32
    xpl = xyz[:, 0, :]
    ypl = xyz[:, 1, :]
    zpl = xyz[:, 2, :]

    ox3, oy3, oz3 = pl.pallas_call(
        _fps_body,
        out_shape=[jax.ShapeDtypeStruct((_S, _B, 1), f32)] * 3,
    )(xpl, ypl, zpl)
    qxt = ox3.reshape(_S, _B)
    qyt = oy3.reshape(_S, _B)
    qzt = oz3.reshape(_S, _B)
    ox = qxt.T
    oy = qyt.T
    oz = qzt.T

    table = pl.pallas_call(
        _table_body,
        grid=(_B, _N // 256),
        in_specs=[
            pl.BlockSpec((1, 128, 256), lambda b, j: (b, 0, j)),
        ],
        out_specs=pl.BlockSpec((256, 128), lambda b, j: (b * (_N // 256) + j, 0)),
        out_shape=jax.ShapeDtypeStruct((_B * _N, 128), f32),
        compiler_params=pltpu.CompilerParams(
            dimension_semantics=("arbitrary", "arbitrary")),
    )(points)

    mesh = plsc.VectorSubcoreMesh(core_axis_name="c", subcore_axis_name="s")
    sc = functools.partial(
        pl.kernel,
        out_type=[
            jax.ShapeDtypeStruct((_M, 128), f32),
            jax.ShapeDtypeStruct((_M, 16), f32),
        ],
        mesh=mesh,
        compiler_params=pltpu.CompilerParams(needs_layout_passes=False),
        scratch_types=[
            pltpu.VMEM((_N,), f32), pltpu.VMEM((_N,), f32),
            pltpu.VMEM((_N,), f32), pltpu.VMEM((_N,), f32),
            pltpu.VMEM((_N,), f32), pltpu.VMEM((_N,), f32),
            pltpu.VMEM((_N,), f32),
            pltpu.VMEM((_RPW,), f32), pltpu.VMEM((_RPW,), f32),
            pltpu.VMEM((_RPW,), f32),
            pltpu.VMEM((48,), jnp.int32),
            pltpu.VMEM((_RPW * _K,), jnp.int32),
            pltpu.VMEM((8 * _K, 16), f32),
            pltpu.VMEM((128, 128), f32), pltpu.VMEM((128, 128), f32),
            pltpu.SemaphoreType.DMA, pltpu.SemaphoreType.DMA,
        ],
    )(_sc_body)
    gfeat, gxyz = sc(xpl.reshape(-1), ypl.reshape(-1), zpl.reshape(-1),
                     ox.reshape(-1), oy.reshape(-1), oz.reshape(-1), table)

    w0h = jnp.pad(W0.T, ((0, _DPAD - 131), (0, 0))).astype(jnp.bfloat16)
    w1h = W1.T.astype(jnp.bfloat16)
    w2h = W2.T.astype(jnp.bfloat16)

    n_blk = _M // _BLK
    cp = pltpu.CompilerParams(dimension_semantics=("arbitrary",))
    y0, st0 = pl.pallas_call(
        _mlp0_body,
        grid=(n_blk,),
        in_specs=[
            pl.BlockSpec((_BLK, 128), lambda t: (t, 0)),
            pl.BlockSpec((_BLK, 16), lambda t: (t, 0)),
            pl.BlockSpec((_RQ, 8), lambda t: (t % (_S // _RQ), 0)),
            pl.BlockSpec((_RQ, 8), lambda t: (t % (_S // _RQ), 0)),
            pl.BlockSpec((_RQ, 8), lambda t: (t % (_S // _RQ), 0)),
            pl.BlockSpec((_DPAD, 128), lambda t: (0, 0)),
            pl.BlockSpec((8, 128), lambda t: (0, 0)),
        ],
        out_specs=[
            pl.BlockSpec((_BLK, 128), lambda t: (t, 0)),
            pl.BlockSpec((8, 128), lambda t: (0, 0)),
        ],
        out_shape=[
            jax.ShapeDtypeStruct((_M, 128), f32),
            jax.ShapeDtypeStruct((8, 128), f32),
        ],
        compiler_params=cp,
    )(gfeat, gxyz, qxt, qyt, qzt, w0h, _bc(b0))

    def mid_layer(y, st, g, be, wh, bb, cout):
        return pl.pallas_call(
            _mlp_body,
            grid=(n_blk,),
            in_specs=[
                pl.BlockSpec((_BLK, 128), lambda t: (t, 0)),
                pl.BlockSpec((8, 128), lambda t: (0, 0)),
                pl.BlockSpec((8, 128), lambda t: (0, 0)),
                pl.BlockSpec((8, 128), lambda t: (0, 0)),
                pl.BlockSpec((128, cout), lambda t: (0, 0)),
                pl.BlockSpec((8, cout), lambda t: (0, 0)),
            ],
            out_specs=[
                pl.BlockSpec((_BLK, cout), lambda t: (t, 0)),
                pl.BlockSpec((8, cout), lambda t: (0, 0)),
            ],
            out_shape=[
                jax.ShapeDtypeStruct((_M, cout), f32),
                jax.ShapeDtypeStruct((8, cout), f32),
            ],
            compiler_params=cp,
        )(y, st, _bc(g), _bc(be), wh, _bc(bb))

    y1, st1 = mid_layer(y0, st0, g0, beta0, w1h, b1, 128)
    y2, st2 = mid_layer(y1, st1, g1, beta1, w2h, b2, 256)

    new_points = pl.pallas_call(
        _final_body,
        grid=(_M // 4096,),
        in_specs=[
            pl.BlockSpec((4096, 256), lambda t: (t, 0)),
            pl.BlockSpec((8, 256), lambda t: (0, 0)),
            pl.BlockSpec((8, 256), lambda t: (0, 0)),
            pl.BlockSpec((8, 256), lambda t: (0, 0)),
        ],
        out_specs=pl.BlockSpec((1, 256, 128), lambda t: (t // 8, 0, t % 8)),
        out_shape=jax.ShapeDtypeStruct((_B, 256, _S), f32),
        compiler_params=cp,
    )(y2, st2, _bc(g2), _bc(beta2))

    new_xyz = jnp.stack([ox, oy, oz], axis=1)
    return new_xyz, new_points


# speedup vs baseline: 21.0084x; 1.0576x over previous
import functools

import numpy as np
import jax
import jax.numpy as jnp
from jax import lax
from jax.experimental import pallas as pl
from jax.experimental.pallas import tpu as pltpu
from jax.experimental.pallas import tpu_sc as plsc

_B = 8
_N = 4096
_S = 1024
_K = 32
_DPAD = 144
_EPS = 1e-5
_R2 = np.float32(0.4 ** 2)
_M = _B * _S * _K
_INV_M = np.float32(1.0 / _M)

_BLK = 4096
_RQ = _BLK // _K
_NW = 32
_RPW = _B * _S // _NW
_CPB = _N // 16


def _tree(x, op2, fin):
    n = x.shape[1]
    while n > 128:
        h = n // 2
        x = op2(x[:, :h], x[:, h:])
        n = h
    return fin(x, axis=1, keepdims=True)


def _fps_body(x_ref, y_ref, z_ref, ox_ref, oy_ref, oz_ref):
    X = x_ref[...]
    Y = y_ref[...]
    Z = z_ref[...]
    iota = lax.broadcasted_iota(jnp.int32, (_B, _N), 1)

    def body(i, carry):
        dist, far = carry
        msk = iota == far
        zero = jnp.zeros_like(X)
        cx = _tree(jnp.where(msk, X, zero), jnp.add, jnp.sum)
        cy = _tree(jnp.where(msk, Y, zero), jnp.add, jnp.sum)
        cz = _tree(jnp.where(msk, Z, zero), jnp.add, jnp.sum)
        ox_ref[pl.ds(i, 1)] = cx[None]
        oy_ref[pl.ds(i, 1)] = cy[None]
        oz_ref[pl.ds(i, 1)] = cz[None]
        dx = X - cx
        dy = Y - cy
        dz = Z - cz
        d = (dx * dx + dy * dy) + dz * dz
        dist = jnp.minimum(dist, d)
        m = _tree(dist, jnp.maximum, jnp.max)
        far = _tree(jnp.where(dist == m, iota, _N), jnp.minimum, jnp.min)
        return dist, far

    dist0 = jnp.full((_B, _N), 1e10, jnp.float32)
    far0 = jnp.zeros((_B, 1), jnp.int32)
    lax.fori_loop(0, _S, body, (dist0, far0))


def _table_body(pts_ref, o_ref):
    o_ref[...] = jnp.transpose(pts_ref[0], (1, 0))


def _bf16r(v):
    u = plsc.bitcast(v, jnp.uint32)
    lsb = jnp.bitwise_and(lax.shift_right_logical(u, jnp.uint32(16)),
                          jnp.uint32(1))
    u = u + (jnp.uint32(0x7FFF) + lsb)
    u = jnp.bitwise_and(u, jnp.uint32(0xFFFF0000))
    return plsc.bitcast(u, jnp.float32)


def _sc_body(xf, yf, zf, qxf, qyf, qzf, tab, out, oxyz,
             px, py, pz, pxr, pyr, pzr, sp, qx, qy, qz, slots, gidx, gxb,
             rb0, rb1, sem0, sem1, wsem0, wsem1):
    cid = lax.axis_index("c")
    sid = lax.axis_index("s")
    w = sid * 2 + cid
    b = w // 4
    s0 = (w % 4) * _RPW
    boff = b * _N

    pltpu.sync_copy(xf.at[pl.ds(b * _N, _N)], px)
    pltpu.sync_copy(yf.at[pl.ds(b * _N, _N)], py)
    pltpu.sync_copy(zf.at[pl.ds(b * _N, _N)], pz)
    pltpu.sync_copy(qxf.at[pl.ds(b * _S + s0, _RPW)], qx)
    pltpu.sync_copy(qyf.at[pl.ds(b * _S + s0, _RPW)], qy)
    pltpu.sync_copy(qzf.at[pl.ds(b * _S + s0, _RPW)], qz)

    def spbody(i, _):
        pxv = px[pl.ds(i * 16, 16)]
        pyv = py[pl.ds(i * 16, 16)]
        pzv = pz[pl.ds(i * 16, 16)]
        sp[pl.ds(i * 16, 16)] = (pxv * pxv + pyv * pyv) + pzv * pzv
        pxr[pl.ds(i * 16, 16)] = _bf16r(pxv)
        pyr[pl.ds(i * 16, 16)] = _bf16r(pyv)
        pzr[pl.ds(i * 16, 16)] = _bf16r(pzv)
        return 0

    lax.fori_loop(0, _CPB, spbody, 0)

    iota16 = lax.iota(jnp.int32, 16)

    def _splat(vec, lane):
        zero = jnp.zeros_like(vec)
        s = jnp.sum(jnp.where(iota16 == lane, vec, zero))
        return jnp.full((16,), s, vec.dtype)

    def rowbody(r, _):
        g16 = (r // 16) * 16
        lane = r % 16
        qxs = _splat(qx[pl.ds(g16, 16)], lane)
        qys = _splat(qy[pl.ds(g16, 16)], lane)
        qzs = _splat(qz[pl.ds(g16, 16)], lane)
        sq = (qxs * qxs + qys * qys) + qzs * qzs
        qxr = _bf16r(qxs)
        qyr = _bf16r(qys)
        qzr = _bf16r(qzs)

        def cond(st):
            cnt, c = st
            return (cnt < _K) & (c < _CPB)

        def step(st):
            cnt, c = st
            base = c * 16
            pxv = pxr[pl.ds(base, 16)]
            pyv = pyr[pl.ds(base, 16)]
            pzv = pzr[pl.ds(base, 16)]
            spv = sp[pl.ds(base, 16)]
            m3 = (qxr * pxv + qyr * pyv) + qzr * pzv
            d = (-2.0 * m3 + sq) + spv
            msk = d <= _R2
            mi = msk.astype(jnp.int32)
            slot = (cnt + jnp.cumsum(mi)) - 1
            nvec = base + iota16
            plsc.store_scatter(slots, [slot], nvec, mask=msk)
            return cnt + jnp.sum(mi), c + 1

        cnt, _c = lax.while_loop(cond, step, (jnp.int32(0), jnp.int32(0)))
        first = _splat(slots[pl.ds(0, 16)], 0)
        v0 = jnp.where(iota16 < cnt, slots[pl.ds(0, 16)], first)
        v1 = jnp.where(iota16 + 16 < cnt, slots[pl.ds(16, 16)], first)
        gidx[pl.ds(r * _K, 16)] = v0 + boff
        gidx[pl.ds(r * _K + 16, 16)] = v1 + boff
        r8 = (r % 8) * _K
        for kk, vv in ((0, v0), (1, v1)):
            rows = r8 + 16 * kk + iota16
            for cc, plane in ((0, px), (1, py), (2, pz)):
                coords = plsc.load_gather(plane, [vv])
                plsc.store_scatter(gxb, [rows, jnp.full((16,), cc, jnp.int32)],
                                   coords)
        @pl.when(r % 8 == 7)
        def _():
            pltpu.sync_copy(
                gxb, oxyz.at[pl.ds((w * _RPW + r - 7) * _K, 8 * _K)])
        return 0

    lax.fori_loop(0, _RPW, rowbody, 0)

    ob = w * (_RPW * _K)
    nch = _RPW * _K // 128
    bufs = (rb0, rb1)
    gsems = (sem0, sem1)
    wsems = (wsem0, wsem1)
    gh = [None, None]
    wh = [None, None]
    for j in range(nch):
        p = j % 2
        if j >= 2:
            wh[p].wait()
        gh[p] = pltpu.async_copy(
            tab.at[gidx.at[pl.ds(j * 128, 128)]], bufs[p], gsems[p])
        if j >= 1:
            q = (j - 1) % 2
            gh[q].wait()
            wh[q] = pltpu.async_copy(
                bufs[q], out.at[pl.ds(ob + (j - 1) * 128, 128)], wsems[q])
    gh[(nch - 1) % 2].wait()
    wh[(nch - 1) % 2] = pltpu.async_copy(
        bufs[(nch - 1) % 2], out.at[pl.ds(ob + (nch - 1) * 128, 128)],
        wsems[(nch - 1) % 2])
    wh[0].wait()
    wh[1].wait()


def _mmb(x, wh):
    dn = (((1,), (0,)), ((), ()))
    return lax.dot_general(x.astype(jnp.bfloat16), wh, dn,
                           preferred_element_type=jnp.float32)


def _stats_update(st_ref, y, t):
    @pl.when(t == 0)
    def _():
        st_ref[...] = jnp.zeros_like(st_ref)

    s1 = jnp.sum(y, axis=0).reshape(1, -1)
    s2 = jnp.sum(y * y, axis=0).reshape(1, -1)
    pad = jnp.zeros((6, y.shape[1]), jnp.float32)
    st_ref[...] = st_ref[...] + jnp.concatenate([s1, s2, pad], axis=0)


def _mlp0_body(gf_ref, gx_ref, qx_ref, qy_ref, qz_ref, wh_ref,
               bb_ref, y_ref, st_ref):
    t = pl.program_id(0)
    b = t // (_S * _K // _BLK)
    col = lax.broadcasted_iota(jnp.int32, (_RQ, 8), 1)
    zq = jnp.zeros((_RQ, 8), jnp.float32)

    def pick(ref):
        return jnp.sum(jnp.where(col == b, ref[...], zq), axis=1,
                       keepdims=True)

    x = jnp.concatenate(
        [gx_ref[...][:, 0:3], gf_ref[...],
         jnp.zeros((_RQ * _K, _DPAD - 131), jnp.float32)], axis=1)
    q144 = jnp.concatenate(
        [pick(qx_ref), pick(qy_ref), pick(qz_ref),
         jnp.zeros((_RQ, _DPAD - 3), jnp.float32)], axis=1)
    x3 = x.reshape(_RQ, _K, _DPAD) - q144[:, None, :]
    x2 = x3.reshape(_RQ * _K, _DPAD)
    y = _mmb(x2, wh_ref[...]) + bb_ref[...][0:1, :]
    y_ref[...] = y
    _stats_update(st_ref, y, t)


def _affine_relu(y, st_ref, g_ref, be_ref):
    st = st_ref[...]
    mean = st[0:1, :] * _INV_M
    var = st[1:2, :] * _INV_M - mean * mean
    den = jnp.sqrt(var + _EPS)
    h = (y - mean) / den * g_ref[...][0:1, :] + be_ref[...][0:1, :]
    return jnp.maximum(h, 0.0)


def _mlp_body(y_ref, st_ref, g_ref, be_ref, wh_ref, bb_ref,
              o_ref, sto_ref):
    t = pl.program_id(0)
    h = _affine_relu(y_ref[...], st_ref, g_ref, be_ref)
    o = _mmb(h, wh_ref[...]) + bb_ref[...][0:1, :]
    o_ref[...] = o
    _stats_update(sto_ref, o, t)


def _final_body(y_ref, st_ref, g_ref, be_ref, o_ref):
    h = _affine_relu(y_ref[...], st_ref, g_ref, be_ref)
    mx = jnp.max(h.reshape(128, _K, 256), axis=1)
    o_ref[...] = jnp.transpose(mx, (1, 0))[None]


def _bc(v):
    return jnp.broadcast_to(v.reshape(1, -1), (8, v.shape[0]))


def kernel(xyz, points, W0, b0, g0, beta0, W1, b1, g1, beta1,
           W2, b2, g2, beta2):
    f32 = jnp.float32
    xpl = xyz[:, 0, :]
    ypl = xyz[:, 1, :]
    zpl = xyz[:, 2, :]

    ox3, oy3, oz3 = pl.pallas_call(
        _fps_body,
        out_shape=[jax.ShapeDtypeStruct((_S, _B, 1), f32)] * 3,
    )(xpl, ypl, zpl)
    qxt = ox3.reshape(_S, _B)
    qyt = oy3.reshape(_S, _B)
    qzt = oz3.reshape(_S, _B)
    ox = qxt.T
    oy = qyt.T
    oz = qzt.T

    table = pl.pallas_call(
        _table_body,
        grid=(_B, _N // 1024),
        in_specs=[
            pl.BlockSpec((1, 128, 1024), lambda b, j: (b, 0, j)),
        ],
        out_specs=pl.BlockSpec((1024, 128), lambda b, j: (b * (_N // 1024) + j, 0)),
        out_shape=jax.ShapeDtypeStruct((_B * _N, 128), f32),
        compiler_params=pltpu.CompilerParams(
            dimension_semantics=("arbitrary", "arbitrary")),
    )(points)

    mesh = plsc.VectorSubcoreMesh(core_axis_name="c", subcore_axis_name="s")
    sc = functools.partial(
        pl.kernel,
        out_type=[
            jax.ShapeDtypeStruct((_M, 128), f32),
            jax.ShapeDtypeStruct((_M, 16), f32),
        ],
        mesh=mesh,
        compiler_params=pltpu.CompilerParams(needs_layout_passes=False),
        scratch_types=[
            pltpu.VMEM((_N,), f32), pltpu.VMEM((_N,), f32),
            pltpu.VMEM((_N,), f32), pltpu.VMEM((_N,), f32),
            pltpu.VMEM((_N,), f32), pltpu.VMEM((_N,), f32),
            pltpu.VMEM((_N,), f32),
            pltpu.VMEM((_RPW,), f32), pltpu.VMEM((_RPW,), f32),
            pltpu.VMEM((_RPW,), f32),
            pltpu.VMEM((48,), jnp.int32),
            pltpu.VMEM((_RPW * _K,), jnp.int32),
            pltpu.VMEM((8 * _K, 16), f32),
            pltpu.VMEM((128, 128), f32), pltpu.VMEM((128, 128), f32),
            pltpu.SemaphoreType.DMA, pltpu.SemaphoreType.DMA,
            pltpu.SemaphoreType.DMA, pltpu.SemaphoreType.DMA,
        ],
    )(_sc_body)
    gfeat, gxyz = sc(xpl.reshape(-1), ypl.reshape(-1), zpl.reshape(-1),
                     ox.reshape(-1), oy.reshape(-1), oz.reshape(-1), table)

    w0h = jnp.pad(W0.T, ((0, _DPAD - 131), (0, 0))).astype(jnp.bfloat16)
    w1h = W1.T.astype(jnp.bfloat16)
    w2h = W2.T.astype(jnp.bfloat16)

    n_blk = _M // _BLK
    cp = pltpu.CompilerParams(dimension_semantics=("arbitrary",))
    y0, st0 = pl.pallas_call(
        _mlp0_body,
        grid=(n_blk,),
        in_specs=[
            pl.BlockSpec((_BLK, 128), lambda t: (t, 0)),
            pl.BlockSpec((_BLK, 16), lambda t: (t, 0)),
            pl.BlockSpec((_RQ, 8), lambda t: (t % (_S // _RQ), 0)),
            pl.BlockSpec((_RQ, 8), lambda t: (t % (_S // _RQ), 0)),
            pl.BlockSpec((_RQ, 8), lambda t: (t % (_S // _RQ), 0)),
            pl.BlockSpec((_DPAD, 128), lambda t: (0, 0)),
            pl.BlockSpec((8, 128), lambda t: (0, 0)),
        ],
        out_specs=[
            pl.BlockSpec((_BLK, 128), lambda t: (t, 0)),
            pl.BlockSpec((8, 128), lambda t: (0, 0)),
        ],
        out_shape=[
            jax.ShapeDtypeStruct((_M, 128), f32),
            jax.ShapeDtypeStruct((8, 128), f32),
        ],
        compiler_params=cp,
    )(gfeat, gxyz, qxt, qyt, qzt, w0h, _bc(b0))

    def mid_layer(y, st, g, be, wh, bb, cout):
        return pl.pallas_call(
            _mlp_body,
            grid=(n_blk,),
            in_specs=[
                pl.BlockSpec((_BLK, 128), lambda t: (t, 0)),
                pl.BlockSpec((8, 128), lambda t: (0, 0)),
                pl.BlockSpec((8, 128), lambda t: (0, 0)),
                pl.BlockSpec((8, 128), lambda t: (0, 0)),
                pl.BlockSpec((128, cout), lambda t: (0, 0)),
                pl.BlockSpec((8, cout), lambda t: (0, 0)),
            ],
            out_specs=[
                pl.BlockSpec((_BLK, cout), lambda t: (t, 0)),
                pl.BlockSpec((8, cout), lambda t: (0, 0)),
            ],
            out_shape=[
                jax.ShapeDtypeStruct((_M, cout), f32),
                jax.ShapeDtypeStruct((8, cout), f32),
            ],
            compiler_params=cp,
        )(y, st, _bc(g), _bc(be), wh, _bc(bb))

    y1, st1 = mid_layer(y0, st0, g0, beta0, w1h, b1, 128)
    y2, st2 = mid_layer(y1, st1, g1, beta1, w2h, b2, 256)

    new_points = pl.pallas_call(
        _final_body,
        grid=(_M // 4096,),
        in_specs=[
            pl.BlockSpec((4096, 256), lambda t: (t, 0)),
            pl.BlockSpec((8, 256), lambda t: (0, 0)),
            pl.BlockSpec((8, 256), lambda t: (0, 0)),
            pl.BlockSpec((8, 256), lambda t: (0, 0)),
        ],
        out_specs=pl.BlockSpec((1, 256, 128), lambda t: (t // 8, 0, t % 8)),
        out_shape=jax.ShapeDtypeStruct((_B, 256, _S), f32),
        compiler_params=cp,
    )(y2, st2, _bc(g2), _bc(beta2))

    new_xyz = jnp.stack([ox, oy, oz], axis=1)
    return new_xyz, new_points
